# baseline (speedup 1.0000x reference)
import functools

import jax
import jax.numpy as jnp
from jax import lax
from jax.experimental import pallas as pl
from jax.experimental.pallas import tpu as pltpu
from jax.experimental.pallas import tpu_sc as plsc

_NI = 16384
_NJ = 200
_D = 4
_NW = 32
_TB = 4
_NU = _NJ * _TB
_UPW = _NU // _NW

_mesh = plsc.VectorSubcoreMesh(core_axis_name="c", subcore_axis_name="s")


@functools.partial(
    pl.kernel,
    mesh=_mesh,
    out_type=jax.ShapeDtypeStruct((_NJ, 512, 128), jnp.float32),
    scratch_types=[
        pltpu.VMEM((32,), jnp.float32),
        pltpu.VMEM((2, 32, 128), jnp.int32),
        pltpu.VMEM((2, 128, 128), jnp.float32),
        pltpu.SemaphoreType.DMA,
        pltpu.SemaphoreType.DMA,
        pltpu.SemaphoreType.DMA,
        pltpu.SemaphoreType.DMA,
    ],
    compiler_params=pltpu.CompilerParams(needs_layout_passes=False),
)
def _sc_lookup(idx_hbm, tab_hbm, out_hbm, tab_v, ib, ob, si0, si1, so0, so1):
    wid = lax.axis_index("s") * 2 + lax.axis_index("c")
    pltpu.sync_copy(tab_hbm, tab_v)
    u0 = wid * _UPW
    sin = (si0, si1)
    sout = (so0, so1)

    def idx_src(k):
        u = u0 + k
        j = u // _TB
        return idx_hbm.at[j // 8, pl.ds((u % _TB) * 32, 32), j % 8]

    def out_dst(k):
        u = u0 + k
        j = u // _TB
        return out_hbm.at[j, pl.ds((u % _TB) * 128, 128)]

    def compute(b):
        def t_body(t2):
            for h in range(8):
                g4 = ib[b, t2, pl.ds(h * 16, 16)] * 4
                for c in range(_D):
                    gi = g4 + c if c else g4
                    ob[b, t2 * 4 + c, pl.ds(h * 16, 16)] = (
                        plsc.load_gather(tab_v, [gi]))

        plsc.parallel_loop(0, 32, unroll=4)(t_body)

    pltpu.async_copy(idx_src(0), ib.at[0], sin[0])
    pltpu.async_copy(idx_src(1), ib.at[1], sin[1])

    def pair_body(p, carry):
        for b in (0, 1):
            k = 2 * p + b
            pltpu.make_async_copy(idx_src(k), ib.at[b], sin[b]).wait()

            @pl.when(p >= 1)
            def _wait_out():
                pltpu.make_async_copy(ob.at[b], out_dst(k - 2), sout[b]).wait()

            compute(b)
            pltpu.async_copy(ob.at[b], out_dst(k), sout[b])

            @pl.when(k + 2 < _UPW)
            def _prefetch():
                pltpu.async_copy(idx_src(k + 2), ib.at[b], sin[b])
        return carry

    lax.fori_loop(0, _UPW // 2, pair_body, 0)
    kt = _UPW - 1
    pltpu.make_async_copy(idx_src(kt), ib.at[0], sin[0]).wait()
    pltpu.make_async_copy(ob.at[0], out_dst(kt - 2), sout[0]).wait()
    compute(0)
    pltpu.async_copy(ob.at[0], out_dst(kt), sout[0])
    pltpu.make_async_copy(ob.at[1], out_dst(kt - 1), sout[1]).wait()
    pltpu.make_async_copy(ob.at[0], out_dst(kt), sout[0]).wait()


def kernel(input_ids, embed_table):
    idx = input_ids.astype(jnp.int32)
    idx_phys = idx.reshape(128, 128, 25, 8).transpose(2, 0, 3, 1)
    tab = embed_table.reshape(-1).astype(jnp.float32)
    out = _sc_lookup(idx_phys, tab)
    a = out.reshape(_NJ, 128, _D, 128)
    return a.transpose(1, 3, 0, 2).reshape(_NI, _NJ, _D)


# speedup vs baseline: 1.0042x; 1.0042x over previous
import functools

import jax
import jax.numpy as jnp
from jax import lax
from jax.experimental import pallas as pl
from jax.experimental.pallas import tpu as pltpu
from jax.experimental.pallas import tpu_sc as plsc

_NI = 16384
_NJ = 200
_D = 4
_NW = 32
_TB = 4
_NU = _NJ * _TB
_UPW = _NU // _NW

_mesh = plsc.VectorSubcoreMesh(core_axis_name="c", subcore_axis_name="s")


@functools.partial(
    pl.kernel,
    mesh=_mesh,
    out_type=jax.ShapeDtypeStruct((_NJ, 512, 128), jnp.float32),
    scratch_types=[
        pltpu.VMEM((32,), jnp.float32),
        pltpu.VMEM((2, 32, 128), jnp.int32),
        pltpu.VMEM((2, 128, 128), jnp.float32),
        pltpu.SemaphoreType.DMA,
        pltpu.SemaphoreType.DMA,
        pltpu.SemaphoreType.DMA,
        pltpu.SemaphoreType.DMA,
    ],
    compiler_params=pltpu.CompilerParams(needs_layout_passes=False),
)
def _sc_lookup(idx_hbm, tab_hbm, out_hbm, tab_v, ib, ob, si0, si1, so0, so1):
    wid = lax.axis_index("s") * 2 + lax.axis_index("c")
    pltpu.sync_copy(tab_hbm, tab_v)
    u0 = wid * _UPW
    sin = (si0, si1)
    sout = (so0, so1)

    def idx_src(k):
        u = u0 + k
        j = u // _TB
        return idx_hbm.at[j // 8, pl.ds((u % _TB) * 32, 32), j % 8]

    def out_dst(k):
        u = u0 + k
        j = u // _TB
        return out_hbm.at[j, pl.ds((u % _TB) * 128, 128)]

    def compute(b):
        def t_body(t2):
            for h in range(8):
                g4 = ib[b, t2, pl.ds(h * 16, 16)] * 4
                for c in range(_D):
                    gi = g4 + c if c else g4
                    ob[b, t2 * 4 + c, pl.ds(h * 16, 16)] = (
                        plsc.load_gather(tab_v, [gi]))

        plsc.parallel_loop(0, 32, unroll=8)(t_body)

    pltpu.async_copy(idx_src(0), ib.at[0], sin[0])
    pltpu.async_copy(idx_src(1), ib.at[1], sin[1])

    def pair_body(p, carry):
        for b in (0, 1):
            k = 2 * p + b
            pltpu.make_async_copy(idx_src(k), ib.at[b], sin[b]).wait()

            @pl.when(p >= 1)
            def _wait_out():
                pltpu.make_async_copy(ob.at[b], out_dst(k - 2), sout[b]).wait()

            compute(b)
            pltpu.async_copy(ob.at[b], out_dst(k), sout[b])

            @pl.when(k + 2 < _UPW)
            def _prefetch():
                pltpu.async_copy(idx_src(k + 2), ib.at[b], sin[b])
        return carry

    lax.fori_loop(0, _UPW // 2, pair_body, 0)
    kt = _UPW - 1
    pltpu.make_async_copy(idx_src(kt), ib.at[0], sin[0]).wait()
    pltpu.make_async_copy(ob.at[0], out_dst(kt - 2), sout[0]).wait()
    compute(0)
    pltpu.async_copy(ob.at[0], out_dst(kt), sout[0])
    pltpu.make_async_copy(ob.at[1], out_dst(kt - 1), sout[1]).wait()
    pltpu.make_async_copy(ob.at[0], out_dst(kt), sout[0]).wait()


def kernel(input_ids, embed_table):
    idx = input_ids.astype(jnp.int32)
    idx_phys = idx.reshape(128, 128, 25, 8).transpose(2, 0, 3, 1)
    tab = embed_table.reshape(-1).astype(jnp.float32)
    out = _sc_lookup(idx_phys, tab)
    a = out.reshape(_NJ, 128, _D, 128)
    return a.transpose(1, 3, 0, 2).reshape(_NI, _NJ, _D)


# speedup vs baseline: 1.2899x; 1.2846x over previous
import functools

import jax
import jax.numpy as jnp
from jax import lax
from jax.experimental import pallas as pl
from jax.experimental.pallas import tpu as pltpu
from jax.experimental.pallas import tpu_sc as plsc

_NI = 16384
_NJ = 200
_D = 4
_NW = 32
_TB = 4
_NU = _NJ * _TB
_UPW = _NU // _NW

_mesh = plsc.VectorSubcoreMesh(core_axis_name="c", subcore_axis_name="s")


@functools.partial(
    pl.kernel,
    mesh=_mesh,
    out_type=jax.ShapeDtypeStruct((_NJ, 512, 128), jnp.float32),
    scratch_types=[
        pltpu.VMEM((_D, 16), jnp.float32),
        pltpu.VMEM((2, 32, 128), jnp.int32),
        pltpu.VMEM((2, 128, 128), jnp.float32),
        pltpu.SemaphoreType.DMA,
        pltpu.SemaphoreType.DMA,
        pltpu.SemaphoreType.DMA,
        pltpu.SemaphoreType.DMA,
    ],
    compiler_params=pltpu.CompilerParams(needs_layout_passes=False),
)
def _sc_lookup(idx_hbm, tab_hbm, out_hbm, tab_v, ib, ob, si0, si1, so0, so1):
    wid = lax.axis_index("s") * 2 + lax.axis_index("c")
    pltpu.sync_copy(tab_hbm, tab_v)
    u0 = wid * _UPW
    sin = (si0, si1)
    sout = (so0, so1)

    def idx_src(k):
        u = u0 + k
        j = u // _TB
        return idx_hbm.at[j // 8, pl.ds((u % _TB) * 32, 32), j % 8]

    def out_dst(k):
        u = u0 + k
        j = u // _TB
        return out_hbm.at[j, pl.ds((u % _TB) * 128, 128)]

    tcs = tuple(tab_v[c] for c in range(_D))

    def compute(b):
        def t_body(t2):
            for h in range(8):
                idxv = ib[b, t2, pl.ds(h * 16, 16)]
                for c in range(_D):
                    ob[b, t2 * 4 + c, pl.ds(h * 16, 16)] = (
                        jnp.take_along_axis(tcs[c], idxv, axis=0,
                                            mode="promise_in_bounds"))

        plsc.parallel_loop(0, 32, unroll=8)(t_body)

    pltpu.async_copy(idx_src(0), ib.at[0], sin[0])
    pltpu.async_copy(idx_src(1), ib.at[1], sin[1])

    def pair_body(p, carry):
        for b in (0, 1):
            k = 2 * p + b
            pltpu.make_async_copy(idx_src(k), ib.at[b], sin[b]).wait()

            @pl.when(p >= 1)
            def _wait_out():
                pltpu.make_async_copy(ob.at[b], out_dst(k - 2), sout[b]).wait()

            compute(b)
            pltpu.async_copy(ob.at[b], out_dst(k), sout[b])

            @pl.when(k + 2 < _UPW)
            def _prefetch():
                pltpu.async_copy(idx_src(k + 2), ib.at[b], sin[b])
        return carry

    lax.fori_loop(0, _UPW // 2, pair_body, 0)
    kt = _UPW - 1
    pltpu.make_async_copy(idx_src(kt), ib.at[0], sin[0]).wait()
    pltpu.make_async_copy(ob.at[0], out_dst(kt - 2), sout[0]).wait()
    compute(0)
    pltpu.async_copy(ob.at[0], out_dst(kt), sout[0])
    pltpu.make_async_copy(ob.at[1], out_dst(kt - 1), sout[1]).wait()
    pltpu.make_async_copy(ob.at[0], out_dst(kt), sout[0]).wait()


def kernel(input_ids, embed_table):
    idx = input_ids.astype(jnp.int32)
    idx_phys = idx.reshape(128, 128, 25, 8).transpose(2, 0, 3, 1)
    tab = (jnp.zeros((_D, 16), jnp.float32)
           .at[:, :8].set(embed_table.T.astype(jnp.float32)))
    out = _sc_lookup(idx_phys, tab)
    a = out.reshape(_NJ, 128, _D, 128)
    return a.transpose(1, 3, 0, 2).reshape(_NI, _NJ, _D)


# speedup vs baseline: 1.3462x; 1.0436x over previous
import functools

import jax
import jax.numpy as jnp
from jax import lax
from jax.experimental import pallas as pl
from jax.experimental.pallas import tpu as pltpu
from jax.experimental.pallas import tpu_sc as plsc

_NI = 16384
_NJ = 200
_D = 4
_NW = 32
_TB = 4
_NU = _NJ * _TB
_UPW = _NU // _NW

_mesh = plsc.VectorSubcoreMesh(core_axis_name="c", subcore_axis_name="s")


@functools.partial(
    pl.kernel,
    mesh=_mesh,
    out_type=jax.ShapeDtypeStruct((_NJ, 512, 128), jnp.float32),
    scratch_types=[
        pltpu.VMEM((_D, 16), jnp.float32),
        pltpu.VMEM((3, 32, 128), jnp.int32),
        pltpu.VMEM((3, 128, 128), jnp.float32),
        pltpu.SemaphoreType.DMA,
        pltpu.SemaphoreType.DMA,
        pltpu.SemaphoreType.DMA,
        pltpu.SemaphoreType.DMA,
        pltpu.SemaphoreType.DMA,
        pltpu.SemaphoreType.DMA,
    ],
    compiler_params=pltpu.CompilerParams(needs_layout_passes=False),
)
def _sc_lookup(idx_hbm, tab_hbm, out_hbm, tab_v, ib, ob,
               si0, si1, si2, so0, so1, so2):
    wid = lax.axis_index("s") * 2 + lax.axis_index("c")
    pltpu.sync_copy(tab_hbm, tab_v)
    u0 = wid * _UPW
    sin = (si0, si1, si2)
    sout = (so0, so1, so2)

    def idx_src(k):
        u = u0 + k
        j = u // _TB
        return idx_hbm.at[j // 8, pl.ds((u % _TB) * 32, 32), j % 8]

    def out_dst(k):
        u = u0 + k
        j = u // _TB
        return out_hbm.at[j, pl.ds((u % _TB) * 128, 128)]

    tcs = tuple(tab_v[c] for c in range(_D))

    def compute(b):
        def t_body(t2):
            for h in range(8):
                idxv = ib[b, t2, pl.ds(h * 16, 16)]
                for c in range(_D):
                    ob[b, t2 * 4 + c, pl.ds(h * 16, 16)] = (
                        jnp.take_along_axis(tcs[c], idxv, axis=0,
                                            mode="promise_in_bounds"))

        plsc.parallel_loop(0, 32, unroll=8)(t_body)

    pltpu.async_copy(idx_src(0), ib.at[0], sin[0])
    pltpu.async_copy(idx_src(1), ib.at[1], sin[1])

    def triple_body(p, carry):
        for b in (0, 1, 2):
            k = 3 * p + b
            nb = (b + 2) % 3
            pltpu.make_async_copy(idx_src(k), ib.at[b], sin[b]).wait()

            @pl.when(k + 2 < _UPW)
            def _prefetch():
                pltpu.async_copy(idx_src(k + 2), ib.at[nb], sin[nb])

            @pl.when(p >= 1)
            def _wait_out():
                pltpu.make_async_copy(ob.at[b], out_dst(k - 3), sout[b]).wait()

            compute(b)
            pltpu.async_copy(ob.at[b], out_dst(k), sout[b])
        return carry

    lax.fori_loop(0, _UPW // 3, triple_body, 0)
    kt = _UPW - 1
    bt = kt % 3
    pltpu.make_async_copy(idx_src(kt), ib.at[bt], sin[bt]).wait()
    pltpu.make_async_copy(ob.at[bt], out_dst(kt - 3), sout[bt]).wait()
    compute(bt)
    pltpu.async_copy(ob.at[bt], out_dst(kt), sout[bt])
    for kk in (kt - 2, kt - 1, kt):
        bb = kk % 3
        pltpu.make_async_copy(ob.at[bb], out_dst(kk), sout[bb]).wait()


def kernel(input_ids, embed_table):
    idx = input_ids.astype(jnp.int32)
    idx_phys = idx.reshape(128, 128, 25, 8).transpose(2, 0, 3, 1)
    tab = (jnp.zeros((_D, 16), jnp.float32)
           .at[:, :8].set(embed_table.T.astype(jnp.float32)))
    out = _sc_lookup(idx_phys, tab)
    a = out.reshape(_NJ, 128, _D, 128)
    return a.transpose(1, 3, 0, 2).reshape(_NI, _NJ, _D)
